# baseline (speedup 1.0000x reference)
import jax
import jax.numpy as jnp
from jax.experimental import pallas as pl
from jax.experimental.pallas import tpu as pltpu

B = 32
NCLS = 6
P = 196
D = 768
H = 3072
OUT = 768
ROWS = NCLS + P
HH = H // 2
DQ = 768

_SLOTS = [
    (3, 0, 0), (4, 0, 1),
    (3, 1, 2), (4, 1, 3),
    (3, 2, 4), (4, 2, 5),
    (0, 3, 0), (1, 3, 2), (2, 3, 4),
    (0, 4, 1), (1, 4, 3), (2, 4, 5),
]
_SLOT_GATECOL = [1] * 6 + [0] * 6

_S1 = [
    (3, 6, 3, (0, 2, 4)),
    (4, 9, 3, (1, 3, 5)),
    (0, 0, 2, (6, 9)),
    (1, 2, 2, (7, 10)),
    (2, 4, 2, (8, 11)),
]
_S2_SLOTS = [(0, 1), (2, 3), (4, 5), (6, 7, 8), (9, 10, 11)]


def _bf(v):
    return v.astype(jnp.bfloat16)


def _gelu(v):
    return 0.5 * v * (1.0 + jax.lax.erf(v * 0.7071067811865476))


def _dot_t(a, b):
    return jax.lax.dot_general(
        _bf(a), _bf(b), (((1,), (1,)), ((), ())),
        preferred_element_type=jnp.float32)


def _dot_t_bf(a, b):
    return jax.lax.dot_general(
        a, b, (((1,), (1,)), ((), ())), preferred_element_type=jnp.float32)


def _s1_a(t):
    k = jnp.clip(t, 0, 9) // 2
    return jnp.where(k < 2, k + 3, k - 2)


def _s1_half(t):
    return jnp.clip(t, 0, 9) % 2


def _s2_b(t):
    return jnp.clip(t - 10, 0, 9) // 2


def _s2_k(t):
    return jnp.clip(t - 10, 0, 9) % 2


def _mega_body(x_ref, w1q_ref, w2q_ref, b1_ref, b2_ref, tok_ref, gw_ref,
               a1_ref, a1b_ref, a2_ref, a2b_ref,
               y_ref, clsout_ref,
               w1b_ref, w2b_ref, h_ref, acc_ref, wsel_ref):
    t = pl.program_id(0)

    @pl.when(t == 0)
    def _():
        acc_ref[...] = jnp.zeros_like(acc_ref)
        cols = []
        for s, (a, b, i) in enumerate(_SLOTS):
            g = _dot_t(tok_ref[i], gw_ref[i])
            m = jnp.max(g, axis=-1, keepdims=True)
            e = jnp.exp(g - m)
            sm = e / jnp.sum(e, axis=-1, keepdims=True)
            gk = jnp.min(sm, axis=-1, keepdims=True)
            w = (sm - gk > 0).astype(jnp.float32)
            j = _SLOT_GATECOL[s]
            cols.append(w[:, j:j + 1])
        wsel_ref[...] = jnp.concatenate(cols, axis=1)

    for q in range(4):
        @pl.when(t == q)
        def _(q=q):
            w1b_ref[q * DQ:(q + 1) * DQ, :] = _bf(w1q_ref[...])
            w2b_ref[:, q * DQ:(q + 1) * DQ] = _bf(w2q_ref[...])

    for tt in range(10):
        @pl.when(t == tt)
        def _(tt=tt):
            a, r0, ntok, slots = _S1[tt // 2]
            half = tt % 2
            rows = _bf(tok_ref[r0:r0 + ntok].reshape(ntok * B, D))
            z = _dot_t_bf(rows, a1_ref[0])
            z = z + a1b_ref[0, 0, half * HH:(half + 1) * HH]
            z = _bf(_gelu(z))
            for n, s in enumerate(slots):
                h_ref[s:s + 1, :, half * HH:(half + 1) * HH] = (
                    z[n * B:(n + 1) * B].reshape(1, B, HH))

    @pl.when(t >= 4)
    def _():
        xb = _bf(x_ref[0])
        acc = b2_ref[...]
        for c in range(4):
            z = _dot_t_bf(xb, w1b_ref[c * DQ:(c + 1) * DQ, :])
            z = z + b1_ref[:, c * DQ:(c + 1) * DQ]
            z = _bf(_gelu(z))
            acc = acc + _dot_t_bf(z, w2b_ref[:, c * DQ:(c + 1) * DQ])
        y_ref[0] = acc

    for u in range(10):
        @pl.when(t == 10 + u)
        def _(u=u):
            b, k = u // 2, u % 2
            slots = _S2_SLOTS[b]
            ns = len(slots)
            hrows = h_ref[slots[0]:slots[0] + ns, :,
                          k * HH:(k + 1) * HH].reshape(ns * B, HH)
            o = _dot_t_bf(hrows, _bf(a2_ref[0]))
            if k == 0:
                o = o + a2b_ref[0]
            for n, s in enumerate(slots):
                i = _SLOTS[s][2]
                contrib = o[n * B:(n + 1) * B] * wsel_ref[:, s:s + 1]
                acc_ref[i:i + 1] = acc_ref[i:i + 1] + contrib.reshape(1, B, OUT)

    @pl.when(t == 35)
    def _():
        clsout_ref[...] = acc_ref[...]


def _clsfill_body(cls_ref, yin_ref, o_ref):
    parts = [cls_ref[i].reshape(B, 1, OUT) for i in range(NCLS)]
    parts.append(yin_ref[:, NCLS:8, :])
    o_ref[...] = jnp.concatenate(parts, axis=1)


def kernel(x, mids, gate_W, moe0_W1, moe0_b1, moe0_W2, moe0_b2,
           atom1_W, atom1_b, atom2_W, atom2_b):
    del mids
    toks = x[:, :NCLS, :].transpose(1, 0, 2)
    tokcat = jnp.concatenate([toks, toks[0::2], toks[1::2]], axis=0)
    b1r = moe0_b1.reshape(1, H)
    b2r = moe0_b2.reshape(1, OUT)
    a1b = atom1_b.reshape(5, 1, H)
    a2b = atom2_b.reshape(5, 1, OUT)

    y0, cls_out = pl.pallas_call(
        _mega_body,
        grid=(36,),
        in_specs=[
            pl.BlockSpec((1, ROWS, D), lambda t: (jnp.clip(t - 4, 0, 31), 0, 0)),
            pl.BlockSpec((DQ, D), lambda t: (jnp.clip(t, 0, 3), 0)),
            pl.BlockSpec((OUT, DQ), lambda t: (0, jnp.clip(t, 0, 3))),
            pl.BlockSpec((1, H), lambda t: (0, 0)),
            pl.BlockSpec((1, OUT), lambda t: (0, 0)),
            pl.BlockSpec((12, B, D), lambda t: (0, 0, 0)),
            pl.BlockSpec((NCLS, 2, D), lambda t: (0, 0, 0)),
            pl.BlockSpec((1, HH, D), lambda t: (_s1_a(t), _s1_half(t), 0)),
            pl.BlockSpec((1, 1, H), lambda t: (_s1_a(t), 0, 0)),
            pl.BlockSpec((1, OUT, HH), lambda t: (_s2_b(t), 0, _s2_k(t))),
            pl.BlockSpec((1, 1, OUT), lambda t: (_s2_b(t), 0, 0)),
        ],
        out_specs=[
            pl.BlockSpec((1, ROWS, OUT), lambda t: (jnp.clip(t - 4, 0, 31), 0, 0)),
            pl.BlockSpec((NCLS, B, OUT), lambda t: (0, 0, 0)),
        ],
        out_shape=[
            jax.ShapeDtypeStruct((B, ROWS, OUT), jnp.float32),
            jax.ShapeDtypeStruct((NCLS, B, OUT), jnp.float32),
        ],
        scratch_shapes=[
            pltpu.VMEM((H, D), jnp.bfloat16),
            pltpu.VMEM((OUT, H), jnp.bfloat16),
            pltpu.VMEM((12, B, H), jnp.bfloat16),
            pltpu.VMEM((NCLS, B, OUT), jnp.float32),
            pltpu.VMEM((B, 12), jnp.float32),
        ],
    )(x, moe0_W1, moe0_W2, b1r, b2r, tokcat, gate_W, atom1_W, a1b,
      atom2_W, a2b)

    y = pl.pallas_call(
        _clsfill_body,
        grid=(1,),
        in_specs=[
            pl.BlockSpec((NCLS, B, OUT), lambda _: (0, 0, 0)),
            pl.BlockSpec((B, 8, OUT), lambda _: (0, 0, 0)),
        ],
        out_specs=pl.BlockSpec((B, 8, OUT), lambda _: (0, 0, 0)),
        out_shape=jax.ShapeDtypeStruct((B, ROWS, OUT), jnp.float32),
        input_output_aliases={1: 0},
    )(cls_out, y0)

    return y


# speedup vs baseline: 1.0590x; 1.0590x over previous
import jax
import jax.numpy as jnp
from jax.experimental import pallas as pl
from jax.experimental.pallas import tpu as pltpu

B = 32
NCLS = 6
P = 196
D = 768
H = 3072
OUT = 768
ROWS = NCLS + P
HH = H // 2
DQ = 768

_SLOTS = [
    (3, 0, 0), (4, 0, 1),
    (3, 1, 2), (4, 1, 3),
    (3, 2, 4), (4, 2, 5),
    (0, 3, 0), (1, 3, 2), (2, 3, 4),
    (0, 4, 1), (1, 4, 3), (2, 4, 5),
]
_SLOT_GATECOL = [1] * 6 + [0] * 6

_S1 = [
    (3, 6, 3, (0, 2, 4)),
    (4, 9, 3, (1, 3, 5)),
    (0, 0, 2, (6, 9)),
    (1, 2, 2, (7, 10)),
    (2, 4, 2, (8, 11)),
]
_S2_SLOTS = [(0, 1), (2, 3), (4, 5), (6, 7, 8), (9, 10, 11)]


def _bf(v):
    return v.astype(jnp.bfloat16)


def _gelu(v):
    return 0.5 * v * (1.0 + jax.lax.erf(v * 0.7071067811865476))


def _dot_t(a, b):
    return jax.lax.dot_general(
        _bf(a), _bf(b), (((1,), (1,)), ((), ())),
        preferred_element_type=jnp.float32)


def _dot_t_bf(a, b):
    return jax.lax.dot_general(
        a, b, (((1,), (1,)), ((), ())), preferred_element_type=jnp.float32)


def _s1_a(t):
    k = jnp.clip(t, 0, 9) // 2
    return jnp.where(k < 2, k + 3, k - 2)


def _s1_half(t):
    return jnp.clip(t, 0, 9) % 2


def _s2_b(t):
    return jnp.clip(t - 10, 0, 9) // 2


def _s2_k(t):
    return jnp.clip(t - 10, 0, 9) % 2


def _mega_body(x_ref, w1q_ref, w2q_ref, b1_ref, b2_ref, tok_ref, gw_ref,
               a1_ref, a1b_ref, a2_ref, a2b_ref,
               y_ref, clsout_ref,
               w1b_ref, w2b_ref, h_ref, acc_ref, wsel_ref):
    t = pl.program_id(0)

    @pl.when(t == 0)
    def _():
        acc_ref[...] = jnp.zeros_like(acc_ref)
        cols = []
        for s, (a, b, i) in enumerate(_SLOTS):
            g = _dot_t(tok_ref[i], gw_ref[i])
            m = jnp.max(g, axis=-1, keepdims=True)
            e = jnp.exp(g - m)
            sm = e / jnp.sum(e, axis=-1, keepdims=True)
            gk = jnp.min(sm, axis=-1, keepdims=True)
            w = (sm - gk > 0).astype(jnp.float32)
            j = _SLOT_GATECOL[s]
            cols.append(w[:, j:j + 1])
        wsel_ref[...] = jnp.concatenate(cols, axis=1)

    for q in range(4):
        @pl.when(t == q)
        def _(q=q):
            w1b_ref[q * DQ:(q + 1) * DQ, :] = _bf(w1q_ref[...])
            w2b_ref[:, q * DQ:(q + 1) * DQ] = _bf(w2q_ref[...])

    for tt in range(10):
        @pl.when(t == tt)
        def _(tt=tt):
            a, r0, ntok, slots = _S1[tt // 2]
            half = tt % 2
            rows = _bf(tok_ref[r0:r0 + ntok].reshape(ntok * B, D))
            z = _dot_t_bf(rows, a1_ref[0])
            z = z + a1b_ref[0, 0, half * HH:(half + 1) * HH]
            z = _bf(_gelu(z))
            for n, s in enumerate(slots):
                h_ref[s:s + 1, :, half * HH:(half + 1) * HH] = (
                    z[n * B:(n + 1) * B].reshape(1, B, HH))

    @pl.when(t >= 4)
    def _():
        for q in range(2):
            xb = _bf(x_ref[q])
            acc = b2_ref[...]
            for c in range(4):
                z = _dot_t_bf(xb, w1b_ref[c * DQ:(c + 1) * DQ, :])
                z = z + b1_ref[:, c * DQ:(c + 1) * DQ]
                z = _bf(_gelu(z))
                acc = acc + _dot_t_bf(z, w2b_ref[:, c * DQ:(c + 1) * DQ])
            y_ref[q] = acc

    for u in range(10):
        @pl.when(t == 10 + u)
        def _(u=u):
            b, k = u // 2, u % 2
            slots = _S2_SLOTS[b]
            ns = len(slots)
            hrows = h_ref[slots[0]:slots[0] + ns, :,
                          k * HH:(k + 1) * HH].reshape(ns * B, HH)
            o = _dot_t_bf(hrows, _bf(a2_ref[0]))
            if k == 0:
                o = o + a2b_ref[0]
            for n, s in enumerate(slots):
                i = _SLOTS[s][2]
                contrib = o[n * B:(n + 1) * B] * wsel_ref[:, s:s + 1]
                acc_ref[i:i + 1] = acc_ref[i:i + 1] + contrib.reshape(1, B, OUT)

    @pl.when(t == 19)
    def _():
        clsout_ref[...] = acc_ref[...]


def _clsfill_body(cls_ref, yin_ref, o_ref):
    parts = [cls_ref[i].reshape(B, 1, OUT) for i in range(NCLS)]
    parts.append(yin_ref[:, NCLS:8, :])
    o_ref[...] = jnp.concatenate(parts, axis=1)


def kernel(x, mids, gate_W, moe0_W1, moe0_b1, moe0_W2, moe0_b2,
           atom1_W, atom1_b, atom2_W, atom2_b):
    del mids
    toks = x[:, :NCLS, :].transpose(1, 0, 2)
    tokcat = jnp.concatenate([toks, toks[0::2], toks[1::2]], axis=0)
    b1r = moe0_b1.reshape(1, H)
    b2r = moe0_b2.reshape(1, OUT)
    a1b = atom1_b.reshape(5, 1, H)
    a2b = atom2_b.reshape(5, 1, OUT)

    y0, cls_out = pl.pallas_call(
        _mega_body,
        grid=(20,),
        in_specs=[
            pl.BlockSpec((2, ROWS, D), lambda t: (jnp.clip(t - 4, 0, 15), 0, 0)),
            pl.BlockSpec((DQ, D), lambda t: (jnp.clip(t, 0, 3), 0)),
            pl.BlockSpec((OUT, DQ), lambda t: (0, jnp.clip(t, 0, 3))),
            pl.BlockSpec((1, H), lambda t: (0, 0)),
            pl.BlockSpec((1, OUT), lambda t: (0, 0)),
            pl.BlockSpec((12, B, D), lambda t: (0, 0, 0)),
            pl.BlockSpec((NCLS, 2, D), lambda t: (0, 0, 0)),
            pl.BlockSpec((1, HH, D), lambda t: (_s1_a(t), _s1_half(t), 0)),
            pl.BlockSpec((1, 1, H), lambda t: (_s1_a(t), 0, 0)),
            pl.BlockSpec((1, OUT, HH), lambda t: (_s2_b(t), 0, _s2_k(t))),
            pl.BlockSpec((1, 1, OUT), lambda t: (_s2_b(t), 0, 0)),
        ],
        out_specs=[
            pl.BlockSpec((2, ROWS, OUT), lambda t: (jnp.clip(t - 4, 0, 15), 0, 0)),
            pl.BlockSpec((NCLS, B, OUT), lambda t: (0, 0, 0)),
        ],
        out_shape=[
            jax.ShapeDtypeStruct((B, ROWS, OUT), jnp.float32),
            jax.ShapeDtypeStruct((NCLS, B, OUT), jnp.float32),
        ],
        scratch_shapes=[
            pltpu.VMEM((H, D), jnp.bfloat16),
            pltpu.VMEM((OUT, H), jnp.bfloat16),
            pltpu.VMEM((12, B, H), jnp.bfloat16),
            pltpu.VMEM((NCLS, B, OUT), jnp.float32),
            pltpu.VMEM((B, 12), jnp.float32),
        ],
    )(x, moe0_W1, moe0_W2, b1r, b2r, tokcat, gate_W, atom1_W, a1b,
      atom2_W, a2b)

    y = pl.pallas_call(
        _clsfill_body,
        grid=(1,),
        in_specs=[
            pl.BlockSpec((NCLS, B, OUT), lambda _: (0, 0, 0)),
            pl.BlockSpec((B, 8, OUT), lambda _: (0, 0, 0)),
        ],
        out_specs=pl.BlockSpec((B, 8, OUT), lambda _: (0, 0, 0)),
        out_shape=jax.ShapeDtypeStruct((B, ROWS, OUT), jnp.float32),
        input_output_aliases={1: 0},
    )(cls_out, y0)

    return y
